# baseline (speedup 1.0000x reference)
import functools

import jax
import jax.numpy as jnp
from jax import lax
from jax.experimental import pallas as pl
from jax.experimental.pallas import tpu as pltpu
from jax.experimental.pallas import tpu_sc as plsc

N = 10000
E = 160000
HALF = 128
CHUNK = 128
NSUB = 16
E_PAD = 163840
EPT = E_PAD // NSUB
CHUNKP = 128
NBODY = 8
BUFS = 2
NOUTER = EPT // (CHUNKP * NBODY)
NBUF = 4
NCHUNK_D = EPT // CHUNK // 2
ACC_ROWS = 10240
RPT = ACC_ROWS // NSUB
DUMMY = N

_mesh = plsc.VectorSubcoreMesh(core_axis_name="c", subcore_axis_name="s")


def _fill_128x128(buf, value):
    nrows = buf.shape[0]

    def body(i, _):
        for j in range(8):
            buf[i, pl.ds(j * 16, 16)] = jnp.full((16,), value, jnp.float32)
        return 0

    lax.fori_loop(0, nrows, body, 0)


def _zero_shared(zbuf, sh, s):
    nrows = zbuf.shape[0]

    def zcp(j, _):
        pltpu.sync_copy(zbuf, sh.at[pl.ds(s * RPT + j * nrows, nrows)])
        return 0

    lax.fori_loop(0, RPT // nrows, zcp, 0)


def _deg_body(dst4_hbm, deg_hbm, dst_all, ones_v, zbuf, deg_sh,
              ssem0, ssem1, ssem2, ssem3):
    c = lax.axis_index("c")
    s = lax.axis_index("s")
    ssems = [ssem0, ssem1, ssem2, ssem3]

    _fill_128x128(zbuf, 0.0)
    _fill_128x128(ones_v, 1.0)
    pltpu.sync_copy(dst4_hbm.at[c, s], dst_all)
    _zero_shared(zbuf, deg_sh, s)
    plsc.subcore_barrier()

    def outer(og, _):
        for b in range(NBUF):
            g = og * NBUF + b
            pltpu.async_copy(ones_v, deg_sh.at[dst_all.at[g]], ssems[b],
                             add=True)
        for b in range(NBUF):
            g = og * NBUF + b
            pltpu.make_async_copy(ones_v, deg_sh.at[dst_all.at[g]],
                                  ssems[b]).wait()
        return 0

    lax.fori_loop(0, NCHUNK_D // NBUF, outer, 0)
    plsc.subcore_barrier()
    pltpu.sync_copy(deg_sh.at[pl.ds(s * RPT, RPT)],
                    deg_hbm.at[c, pl.ds(s * RPT, RPT)])


@functools.partial(
    pl.kernel,
    out_type=jax.ShapeDtypeStruct((2, ACC_ROWS, HALF), jnp.float32),
    mesh=_mesh,
    scratch_types=[
        pltpu.VMEM((NCHUNK_D, CHUNK), jnp.int32),
        pltpu.VMEM((CHUNK, HALF), jnp.float32),
        pltpu.VMEM((CHUNK, HALF), jnp.float32),
        pltpu.VMEM_SHARED((ACC_ROWS, HALF), jnp.float32),
        pltpu.SemaphoreType.DMA,
        pltpu.SemaphoreType.DMA,
        pltpu.SemaphoreType.DMA,
        pltpu.SemaphoreType.DMA,
    ],
    name="gcn_sc_degree",
)
def _deg_kernel(dst4_hbm, deg_hbm, dst_all, ones_v, zbuf, deg_sh,
                ssem0, ssem1, ssem2, ssem3):
    _deg_body(dst4_hbm, deg_hbm, dst_all, ones_v, zbuf, deg_sh,
              ssem0, ssem1, ssem2, ssem3)


def _prop_body(yil_hbm, comb_hbm, out_hbm,
               idx_blk, rows0, rows1, acc_sh, *sems):
    c = lax.axis_index("c")
    s = lax.axis_index("s")
    rows = [rows0, rows1]
    gsems = list(sems[:BUFS])
    ssems = list(sems[BUFS:])

    _fill_128x128(rows0, 0.0)
    _zero_shared(rows0, acc_sh, s)
    plsc.subcore_barrier()

    def _gat(k):
        b = k % BUFS
        return pltpu.make_async_copy(yil_hbm.at[idx_blk.at[0, k]], rows[b],
                                     gsems[b])

    def _scat(k):
        b = k % BUFS
        return pltpu.make_async_copy(rows[b], acc_sh.at[idx_blk.at[1, k]],
                                     ssems[b])

    def outer(og, _):
        pltpu.sync_copy(comb_hbm.at[c, s, og], idx_blk)
        for k in range(BUFS):
            pltpu.async_copy(yil_hbm.at[idx_blk.at[0, k]], rows[k], gsems[k])
        for w in range(NBODY // BUFS):
            for b in range(BUFS):
                k = w * BUFS + b
                _gat(k).wait()
                pltpu.async_copy(rows[b], acc_sh.at[idx_blk.at[1, k]],
                                 ssems[b], add=True)
            if w < NBODY // BUFS - 1:
                for b in range(BUFS):
                    k = w * BUFS + b
                    _scat(k).wait()
                    pltpu.async_copy(yil_hbm.at[idx_blk.at[0, k + BUFS]],
                                     rows[b], gsems[b])
        for k in range(NBODY - BUFS, NBODY):
            _scat(k).wait()
        return 0

    lax.fori_loop(0, NOUTER, outer, 0)
    plsc.subcore_barrier()
    pltpu.sync_copy(acc_sh.at[pl.ds(s * RPT, RPT)],
                    out_hbm.at[c, pl.ds(s * RPT, RPT)])


@functools.partial(
    pl.kernel,
    out_type=jax.ShapeDtypeStruct((2, ACC_ROWS, HALF), jnp.float32),
    mesh=_mesh,
    scratch_types=[
        pltpu.VMEM((2, NBODY, CHUNKP), jnp.int32),
        pltpu.VMEM((CHUNKP, HALF), jnp.float32),
        pltpu.VMEM((CHUNKP, HALF), jnp.float32),
        pltpu.VMEM_SHARED((ACC_ROWS, HALF), jnp.float32),
        pltpu.SemaphoreType.DMA,
        pltpu.SemaphoreType.DMA,
        pltpu.SemaphoreType.DMA,
        pltpu.SemaphoreType.DMA,
    ],
    name="gcn_sc_propagate",
)
def _prop_kernel(yil_hbm, comb_hbm, out_hbm,
                 idx_blk, rows0, rows1, acc_sh, *sems):
    _prop_body(yil_hbm, comb_hbm, out_hbm,
               idx_blk, rows0, rows1, acc_sh, *sems)


RB = 400
NRB = N // RB


def _dinv(d0_blk, d1_blk):
    return lax.rsqrt(d0_blk[:, 0:1] + d1_blk[:, 0:1] + 1.0)


def _prep_tc(x_ref, d0_ref, d1_ref, y_ref):
    d = _dinv(d0_ref[...], d1_ref[...])
    y_ref[...] = x_ref[...] * d


def _mid_tc(x_ref, a0_ref, a1_ref, d0_ref, d1_ref, w1t_ref, b1_ref, w2t_ref,
            z_ref, y2_ref):
    d = _dinv(d0_ref[...], d1_ref[...])
    acc = jnp.concatenate([a0_ref[...], a1_ref[...]], axis=1)
    t1 = d * acc + (d * d) * x_ref[...]
    h = jnp.maximum(
        jnp.dot(t1, w1t_ref[...], preferred_element_type=jnp.float32)
        + b1_ref[...], 0.0)
    z = jnp.dot(h, w2t_ref[...], preferred_element_type=jnp.float32)
    z_ref[...] = z
    y2_ref[...] = d * z


def _final_tc(z_ref, a0_ref, a1_ref, d0_ref, d1_ref, b2_ref, out_ref):
    d = _dinv(d0_ref[...], d1_ref[...])
    acc = jnp.concatenate([a0_ref[...], a1_ref[...]], axis=1)
    o = d * acc + (d * d) * z_ref[...] + b2_ref[...]
    m = jnp.max(o, axis=1, keepdims=True)
    lse = jnp.log(jnp.sum(jnp.exp(o - m), axis=1, keepdims=True)) + m
    out_ref[...] = o - lse


def _row_spec(w):
    return pl.BlockSpec((RB, w), lambda i: (i, 0))


def _full_spec(shape):
    return pl.BlockSpec(shape, lambda i: (0,) * len(shape))


def kernel(input_feature, edge_index, W1, b1, W2, b2):
    x = input_feature
    src = edge_index[0].astype(jnp.int32)
    dst = edge_index[1].astype(jnp.int32)
    pad = E_PAD - E
    srcp = jnp.concatenate([src, jnp.zeros((pad,), jnp.int32)])
    dstp = jnp.concatenate([dst, jnp.full((pad,), DUMMY, jnp.int32)])
    dstv = dstp.reshape(NSUB, NOUTER, NBODY, CHUNKP)
    comb = jnp.stack([
        jnp.stack([(srcp * 2).reshape(dstv.shape), dstv], axis=2),
        jnp.stack([(srcp * 2 + 1).reshape(dstv.shape), dstv], axis=2)])
    dstd = dstp.reshape(2, NSUB, NCHUNK_D, CHUNK)

    deg_raw = _deg_kernel(dstd)
    d0, d1 = deg_raw[0, :N], deg_raw[1, :N]

    y1_ = pl.pallas_call(
        _prep_tc,
        grid=(NRB,),
        in_specs=[_row_spec(2 * HALF), _row_spec(HALF), _row_spec(HALF)],
        out_specs=_row_spec(2 * HALF),
        out_shape=jax.ShapeDtypeStruct((N, 2 * HALF), jnp.float32),
    )(x, d0, d1)

    a1 = _prop_kernel(y1_.reshape(2 * N, HALF), comb)

    z, y2_ = pl.pallas_call(
        _mid_tc,
        grid=(NRB,),
        in_specs=[_row_spec(2 * HALF), _row_spec(HALF), _row_spec(HALF),
                  _row_spec(HALF), _row_spec(HALF),
                  _full_spec((256, 512)), _full_spec((1, 512)),
                  _full_spec((512, 256))],
        out_specs=[_row_spec(2 * HALF), _row_spec(2 * HALF)],
        out_shape=(jax.ShapeDtypeStruct((N, 2 * HALF), jnp.float32),
                   jax.ShapeDtypeStruct((N, 2 * HALF), jnp.float32)),
    )(x, a1[0, :N], a1[1, :N], d0, d1, W1.T, b1.reshape(1, -1), W2.T)

    a2 = _prop_kernel(y2_.reshape(2 * N, HALF), comb)

    out = pl.pallas_call(
        _final_tc,
        grid=(NRB,),
        in_specs=[_row_spec(2 * HALF), _row_spec(HALF), _row_spec(HALF),
                  _row_spec(HALF), _row_spec(HALF), _full_spec((1, 256))],
        out_specs=_row_spec(2 * HALF),
        out_shape=jax.ShapeDtypeStruct((N, 2 * HALF), jnp.float32),
    )(z, a2[0, :N], a2[1, :N], d0, d1, b2.reshape(1, -1))

    return out


# speedup vs baseline: 1.0605x; 1.0605x over previous
import functools

import jax
import jax.numpy as jnp
from jax import lax
from jax.experimental import pallas as pl
from jax.experimental.pallas import tpu as pltpu
from jax.experimental.pallas import tpu_sc as plsc

N = 10000
E = 160000
HALF = 128
CHUNK = 128
NSUB = 16
E_PAD = 163840
EPT = E_PAD // NSUB
CHUNKP = 80
NBODY = 16
BUFS = 4
NOUTER = EPT // (CHUNKP * NBODY)
NBUF = 4
NCHUNK_D = EPT // CHUNK // 2
ACC_ROWS = 10240
RPT = ACC_ROWS // NSUB
DUMMY = N

_mesh = plsc.VectorSubcoreMesh(core_axis_name="c", subcore_axis_name="s")


def _fill_128x128(buf, value):
    nrows = buf.shape[0]

    def body(i, _):
        for j in range(8):
            buf[i, pl.ds(j * 16, 16)] = jnp.full((16,), value, jnp.float32)
        return 0

    lax.fori_loop(0, nrows, body, 0)


def _zero_shared(zbuf, sh, s):
    nrows = zbuf.shape[0]

    def zcp(j, _):
        pltpu.sync_copy(zbuf, sh.at[pl.ds(s * RPT + j * nrows, nrows)])
        return 0

    lax.fori_loop(0, RPT // nrows, zcp, 0)


def _deg_body(dst4_hbm, deg_hbm, dst_all, ones_v, zbuf, deg_sh,
              ssem0, ssem1, ssem2, ssem3):
    c = lax.axis_index("c")
    s = lax.axis_index("s")
    ssems = [ssem0, ssem1, ssem2, ssem3]

    _fill_128x128(zbuf, 0.0)
    _fill_128x128(ones_v, 1.0)
    pltpu.sync_copy(dst4_hbm.at[c, s], dst_all)
    _zero_shared(zbuf, deg_sh, s)
    plsc.subcore_barrier()

    def outer(og, _):
        for b in range(NBUF):
            g = og * NBUF + b
            pltpu.async_copy(ones_v, deg_sh.at[dst_all.at[g]], ssems[b],
                             add=True)
        for b in range(NBUF):
            g = og * NBUF + b
            pltpu.make_async_copy(ones_v, deg_sh.at[dst_all.at[g]],
                                  ssems[b]).wait()
        return 0

    lax.fori_loop(0, NCHUNK_D // NBUF, outer, 0)
    plsc.subcore_barrier()
    pltpu.sync_copy(deg_sh.at[pl.ds(s * RPT, RPT)],
                    deg_hbm.at[c, pl.ds(s * RPT, RPT)])


@functools.partial(
    pl.kernel,
    out_type=jax.ShapeDtypeStruct((2, ACC_ROWS, HALF), jnp.float32),
    mesh=_mesh,
    scratch_types=[
        pltpu.VMEM((NCHUNK_D, CHUNK), jnp.int32),
        pltpu.VMEM((CHUNK, HALF), jnp.float32),
        pltpu.VMEM((CHUNK, HALF), jnp.float32),
        pltpu.VMEM_SHARED((ACC_ROWS, HALF), jnp.float32),
        pltpu.SemaphoreType.DMA,
        pltpu.SemaphoreType.DMA,
        pltpu.SemaphoreType.DMA,
        pltpu.SemaphoreType.DMA,
    ],
    name="gcn_sc_degree",
)
def _deg_kernel(dst4_hbm, deg_hbm, dst_all, ones_v, zbuf, deg_sh,
                ssem0, ssem1, ssem2, ssem3):
    _deg_body(dst4_hbm, deg_hbm, dst_all, ones_v, zbuf, deg_sh,
              ssem0, ssem1, ssem2, ssem3)


def _prop_body(yil_hbm, comb_hbm, out_hbm,
               idx_blk, rows0, rows1, rows2, rows3, acc_sh, *sems):
    c = lax.axis_index("c")
    s = lax.axis_index("s")
    rows = [rows0, rows1, rows2, rows3]
    gsems = list(sems[:BUFS])
    ssems = list(sems[BUFS:])

    _fill_128x128(rows0, 0.0)
    _zero_shared(rows0, acc_sh, s)
    plsc.subcore_barrier()

    def _gat(k):
        b = k % BUFS
        return pltpu.make_async_copy(yil_hbm.at[idx_blk.at[0, k]], rows[b],
                                     gsems[b])

    def _scat(k):
        b = k % BUFS
        return pltpu.make_async_copy(rows[b], acc_sh.at[idx_blk.at[1, k]],
                                     ssems[b])

    def outer(og, _):
        pltpu.sync_copy(comb_hbm.at[c, s, og], idx_blk)
        for k in range(BUFS):
            pltpu.async_copy(yil_hbm.at[idx_blk.at[0, k]], rows[k], gsems[k])
        for w in range(NBODY // BUFS):
            for b in range(BUFS):
                k = w * BUFS + b
                _gat(k).wait()
                pltpu.async_copy(rows[b], acc_sh.at[idx_blk.at[1, k]],
                                 ssems[b], add=True)
            if w < NBODY // BUFS - 1:
                for b in range(BUFS):
                    k = w * BUFS + b
                    _scat(k).wait()
                    pltpu.async_copy(yil_hbm.at[idx_blk.at[0, k + BUFS]],
                                     rows[b], gsems[b])
        for k in range(NBODY - BUFS, NBODY):
            _scat(k).wait()
        return 0

    lax.fori_loop(0, NOUTER, outer, 0)
    plsc.subcore_barrier()
    pltpu.sync_copy(acc_sh.at[pl.ds(s * RPT, RPT)],
                    out_hbm.at[c, pl.ds(s * RPT, RPT)])


@functools.partial(
    pl.kernel,
    out_type=jax.ShapeDtypeStruct((2, ACC_ROWS, HALF), jnp.float32),
    mesh=_mesh,
    scratch_types=[
        pltpu.VMEM((2, NBODY, CHUNKP), jnp.int32),
        pltpu.VMEM((CHUNKP, HALF), jnp.float32),
        pltpu.VMEM((CHUNKP, HALF), jnp.float32),
        pltpu.VMEM((CHUNKP, HALF), jnp.float32),
        pltpu.VMEM((CHUNKP, HALF), jnp.float32),
        pltpu.VMEM_SHARED((ACC_ROWS, HALF), jnp.float32),
        pltpu.SemaphoreType.DMA,
        pltpu.SemaphoreType.DMA,
        pltpu.SemaphoreType.DMA,
        pltpu.SemaphoreType.DMA,
        pltpu.SemaphoreType.DMA,
        pltpu.SemaphoreType.DMA,
        pltpu.SemaphoreType.DMA,
        pltpu.SemaphoreType.DMA,
    ],
    name="gcn_sc_propagate",
)
def _prop_kernel(yil_hbm, comb_hbm, out_hbm,
                 idx_blk, rows0, rows1, rows2, rows3, acc_sh, *sems):
    _prop_body(yil_hbm, comb_hbm, out_hbm,
               idx_blk, rows0, rows1, rows2, rows3, acc_sh, *sems)


RB = 400
NRB = N // RB


def _dinv(d0_blk, d1_blk):
    return lax.rsqrt(d0_blk[:, 0:1] + d1_blk[:, 0:1] + 1.0)


def _prep_tc(x_ref, d0_ref, d1_ref, y_ref):
    d = _dinv(d0_ref[...], d1_ref[...])
    y_ref[...] = x_ref[...] * d


def _mid_tc(x_ref, a0_ref, a1_ref, d0_ref, d1_ref, w1t_ref, b1_ref, w2t_ref,
            z_ref, y2_ref):
    d = _dinv(d0_ref[...], d1_ref[...])
    acc = jnp.concatenate([a0_ref[...], a1_ref[...]], axis=1)
    t1 = d * acc + (d * d) * x_ref[...]
    h = jnp.maximum(
        jnp.dot(t1, w1t_ref[...], preferred_element_type=jnp.float32)
        + b1_ref[...], 0.0)
    z = jnp.dot(h, w2t_ref[...], preferred_element_type=jnp.float32)
    z_ref[...] = z
    y2_ref[...] = d * z


def _final_tc(z_ref, a0_ref, a1_ref, d0_ref, d1_ref, b2_ref, out_ref):
    d = _dinv(d0_ref[...], d1_ref[...])
    acc = jnp.concatenate([a0_ref[...], a1_ref[...]], axis=1)
    o = d * acc + (d * d) * z_ref[...] + b2_ref[...]
    m = jnp.max(o, axis=1, keepdims=True)
    lse = jnp.log(jnp.sum(jnp.exp(o - m), axis=1, keepdims=True)) + m
    out_ref[...] = o - lse


def _row_spec(w):
    return pl.BlockSpec((RB, w), lambda i: (i, 0))


def _full_spec(shape):
    return pl.BlockSpec(shape, lambda i: (0,) * len(shape))


def kernel(input_feature, edge_index, W1, b1, W2, b2):
    x = input_feature
    src = edge_index[0].astype(jnp.int32)
    dst = edge_index[1].astype(jnp.int32)
    pad = E_PAD - E
    srcp = jnp.concatenate([src, jnp.zeros((pad,), jnp.int32)])
    dstp = jnp.concatenate([dst, jnp.full((pad,), DUMMY, jnp.int32)])
    dstv = dstp.reshape(NSUB, NOUTER, NBODY, CHUNKP)
    comb = jnp.stack([
        jnp.stack([(srcp * 2).reshape(dstv.shape), dstv], axis=2),
        jnp.stack([(srcp * 2 + 1).reshape(dstv.shape), dstv], axis=2)])
    dstd = dstp.reshape(2, NSUB, NCHUNK_D, CHUNK)

    deg_raw = _deg_kernel(dstd)
    d0, d1 = deg_raw[0, :N], deg_raw[1, :N]

    y1_ = pl.pallas_call(
        _prep_tc,
        grid=(NRB,),
        in_specs=[_row_spec(2 * HALF), _row_spec(HALF), _row_spec(HALF)],
        out_specs=_row_spec(2 * HALF),
        out_shape=jax.ShapeDtypeStruct((N, 2 * HALF), jnp.float32),
    )(x, d0, d1)

    a1 = _prop_kernel(y1_.reshape(2 * N, HALF), comb)

    z, y2_ = pl.pallas_call(
        _mid_tc,
        grid=(NRB,),
        in_specs=[_row_spec(2 * HALF), _row_spec(HALF), _row_spec(HALF),
                  _row_spec(HALF), _row_spec(HALF),
                  _full_spec((256, 512)), _full_spec((1, 512)),
                  _full_spec((512, 256))],
        out_specs=[_row_spec(2 * HALF), _row_spec(2 * HALF)],
        out_shape=(jax.ShapeDtypeStruct((N, 2 * HALF), jnp.float32),
                   jax.ShapeDtypeStruct((N, 2 * HALF), jnp.float32)),
    )(x, a1[0, :N], a1[1, :N], d0, d1, W1.T, b1.reshape(1, -1), W2.T)

    a2 = _prop_kernel(y2_.reshape(2 * N, HALF), comb)

    out = pl.pallas_call(
        _final_tc,
        grid=(NRB,),
        in_specs=[_row_spec(2 * HALF), _row_spec(HALF), _row_spec(HALF),
                  _row_spec(HALF), _row_spec(HALF), _full_spec((1, 256))],
        out_specs=_row_spec(2 * HALF),
        out_shape=jax.ShapeDtypeStruct((N, 2 * HALF), jnp.float32),
    )(z, a2[0, :N], a2[1, :N], d0, d1, b2.reshape(1, -1))

    return out


# speedup vs baseline: 1.0739x; 1.0126x over previous
import functools

import jax
import jax.numpy as jnp
from jax import lax
from jax.experimental import pallas as pl
from jax.experimental.pallas import tpu as pltpu
from jax.experimental.pallas import tpu_sc as plsc

N = 10000
E = 160000
HALF = 128
CHUNK = 128
NSUB = 16
E_PAD = 163840
EPT = E_PAD // NSUB
CHUNKP = 80
NBODY = 32
BUFS = 4
NOUTER = EPT // (CHUNKP * NBODY)
NBUF = 4
NCHUNK_D = EPT // CHUNK // 2
ACC_ROWS = 10240
RPT = ACC_ROWS // NSUB
DUMMY = N

_mesh = plsc.VectorSubcoreMesh(core_axis_name="c", subcore_axis_name="s")


def _fill_128x128(buf, value):
    nrows = buf.shape[0]

    def body(i, _):
        for j in range(8):
            buf[i, pl.ds(j * 16, 16)] = jnp.full((16,), value, jnp.float32)
        return 0

    lax.fori_loop(0, nrows, body, 0)


def _zero_shared(zbuf, sh, s):
    nrows = zbuf.shape[0]

    def zcp(j, _):
        pltpu.sync_copy(zbuf, sh.at[pl.ds(s * RPT + j * nrows, nrows)])
        return 0

    lax.fori_loop(0, RPT // nrows, zcp, 0)


def _deg_body(dst4_hbm, deg_hbm, dst_all, ones_v, zbuf, deg_sh,
              ssem0, ssem1, ssem2, ssem3):
    c = lax.axis_index("c")
    s = lax.axis_index("s")
    ssems = [ssem0, ssem1, ssem2, ssem3]

    _fill_128x128(zbuf, 0.0)
    _fill_128x128(ones_v, 1.0)
    pltpu.sync_copy(dst4_hbm.at[c, s], dst_all)
    _zero_shared(zbuf, deg_sh, s)
    plsc.subcore_barrier()

    def outer(og, _):
        for b in range(NBUF):
            g = og * NBUF + b
            pltpu.async_copy(ones_v, deg_sh.at[dst_all.at[g]], ssems[b],
                             add=True)
        for b in range(NBUF):
            g = og * NBUF + b
            pltpu.make_async_copy(ones_v, deg_sh.at[dst_all.at[g]],
                                  ssems[b]).wait()
        return 0

    lax.fori_loop(0, NCHUNK_D // NBUF, outer, 0)
    plsc.subcore_barrier()
    pltpu.sync_copy(deg_sh.at[pl.ds(s * RPT, RPT)],
                    deg_hbm.at[c, pl.ds(s * RPT, RPT)])


@functools.partial(
    pl.kernel,
    out_type=jax.ShapeDtypeStruct((2, ACC_ROWS, HALF), jnp.float32),
    mesh=_mesh,
    scratch_types=[
        pltpu.VMEM((NCHUNK_D, CHUNK), jnp.int32),
        pltpu.VMEM((CHUNK, HALF), jnp.float32),
        pltpu.VMEM((CHUNK, HALF), jnp.float32),
        pltpu.VMEM_SHARED((ACC_ROWS, HALF), jnp.float32),
        pltpu.SemaphoreType.DMA,
        pltpu.SemaphoreType.DMA,
        pltpu.SemaphoreType.DMA,
        pltpu.SemaphoreType.DMA,
    ],
    name="gcn_sc_degree",
)
def _deg_kernel(dst4_hbm, deg_hbm, dst_all, ones_v, zbuf, deg_sh,
                ssem0, ssem1, ssem2, ssem3):
    _deg_body(dst4_hbm, deg_hbm, dst_all, ones_v, zbuf, deg_sh,
              ssem0, ssem1, ssem2, ssem3)


def _prop_body(yil_hbm, comb_hbm, out_hbm,
               idx_blk, rows0, rows1, rows2, rows3, acc_sh, *sems):
    c = lax.axis_index("c")
    s = lax.axis_index("s")
    rows = [rows0, rows1, rows2, rows3]
    gsems = list(sems[:BUFS])
    ssems = list(sems[BUFS:])

    _fill_128x128(rows0, 0.0)
    _zero_shared(rows0, acc_sh, s)
    plsc.subcore_barrier()

    def _gat(k):
        b = k % BUFS
        return pltpu.make_async_copy(yil_hbm.at[idx_blk.at[0, k]], rows[b],
                                     gsems[b])

    def _scat(k):
        b = k % BUFS
        return pltpu.make_async_copy(rows[b], acc_sh.at[idx_blk.at[1, k]],
                                     ssems[b])

    def outer(og, _):
        pltpu.sync_copy(comb_hbm.at[c, s, og], idx_blk)
        for k in range(BUFS):
            pltpu.async_copy(yil_hbm.at[idx_blk.at[0, k]], rows[k], gsems[k])
        for w in range(NBODY // BUFS):
            for b in range(BUFS):
                k = w * BUFS + b
                _gat(k).wait()
                pltpu.async_copy(rows[b], acc_sh.at[idx_blk.at[1, k]],
                                 ssems[b], add=True)
            if w < NBODY // BUFS - 1:
                for b in range(BUFS):
                    k = w * BUFS + b
                    _scat(k).wait()
                    pltpu.async_copy(yil_hbm.at[idx_blk.at[0, k + BUFS]],
                                     rows[b], gsems[b])
        for k in range(NBODY - BUFS, NBODY):
            _scat(k).wait()
        return 0

    lax.fori_loop(0, NOUTER, outer, 0)
    plsc.subcore_barrier()
    pltpu.sync_copy(acc_sh.at[pl.ds(s * RPT, RPT)],
                    out_hbm.at[c, pl.ds(s * RPT, RPT)])


@functools.partial(
    pl.kernel,
    out_type=jax.ShapeDtypeStruct((2, ACC_ROWS, HALF), jnp.float32),
    mesh=_mesh,
    scratch_types=[
        pltpu.VMEM((2, NBODY, CHUNKP), jnp.int32),
        pltpu.VMEM((CHUNKP, HALF), jnp.float32),
        pltpu.VMEM((CHUNKP, HALF), jnp.float32),
        pltpu.VMEM((CHUNKP, HALF), jnp.float32),
        pltpu.VMEM((CHUNKP, HALF), jnp.float32),
        pltpu.VMEM_SHARED((ACC_ROWS, HALF), jnp.float32),
        pltpu.SemaphoreType.DMA,
        pltpu.SemaphoreType.DMA,
        pltpu.SemaphoreType.DMA,
        pltpu.SemaphoreType.DMA,
        pltpu.SemaphoreType.DMA,
        pltpu.SemaphoreType.DMA,
        pltpu.SemaphoreType.DMA,
        pltpu.SemaphoreType.DMA,
    ],
    name="gcn_sc_propagate",
)
def _prop_kernel(yil_hbm, comb_hbm, out_hbm,
                 idx_blk, rows0, rows1, rows2, rows3, acc_sh, *sems):
    _prop_body(yil_hbm, comb_hbm, out_hbm,
               idx_blk, rows0, rows1, rows2, rows3, acc_sh, *sems)


RB = 400
NRB = N // RB


def _dinv(d0_blk, d1_blk):
    return lax.rsqrt(d0_blk[:, 0:1] + d1_blk[:, 0:1] + 1.0)


def _prep_tc(x_ref, d0_ref, d1_ref, y_ref):
    d = _dinv(d0_ref[...], d1_ref[...])
    y_ref[...] = x_ref[...] * d


def _mid_tc(x_ref, a0_ref, a1_ref, d0_ref, d1_ref, w1t_ref, b1_ref, w2t_ref,
            z_ref, y2_ref):
    d = _dinv(d0_ref[...], d1_ref[...])
    acc = jnp.concatenate([a0_ref[...], a1_ref[...]], axis=1)
    t1 = d * acc + (d * d) * x_ref[...]
    h = jnp.maximum(
        jnp.dot(t1, w1t_ref[...], preferred_element_type=jnp.float32)
        + b1_ref[...], 0.0)
    z = jnp.dot(h, w2t_ref[...], preferred_element_type=jnp.float32)
    z_ref[...] = z
    y2_ref[...] = d * z


def _final_tc(z_ref, a0_ref, a1_ref, d0_ref, d1_ref, b2_ref, out_ref):
    d = _dinv(d0_ref[...], d1_ref[...])
    acc = jnp.concatenate([a0_ref[...], a1_ref[...]], axis=1)
    o = d * acc + (d * d) * z_ref[...] + b2_ref[...]
    m = jnp.max(o, axis=1, keepdims=True)
    lse = jnp.log(jnp.sum(jnp.exp(o - m), axis=1, keepdims=True)) + m
    out_ref[...] = o - lse


def _row_spec(w):
    return pl.BlockSpec((RB, w), lambda i: (i, 0))


def _full_spec(shape):
    return pl.BlockSpec(shape, lambda i: (0,) * len(shape))


def kernel(input_feature, edge_index, W1, b1, W2, b2):
    x = input_feature
    src = edge_index[0].astype(jnp.int32)
    dst = edge_index[1].astype(jnp.int32)
    pad = E_PAD - E
    srcp = jnp.concatenate([src, jnp.zeros((pad,), jnp.int32)])
    dstp = jnp.concatenate([dst, jnp.full((pad,), DUMMY, jnp.int32)])
    dstv = dstp.reshape(NSUB, NOUTER, NBODY, CHUNKP)
    comb = jnp.stack([
        jnp.stack([(srcp * 2).reshape(dstv.shape), dstv], axis=2),
        jnp.stack([(srcp * 2 + 1).reshape(dstv.shape), dstv], axis=2)])
    dstd = dstp.reshape(2, NSUB, NCHUNK_D, CHUNK)

    deg_raw = _deg_kernel(dstd)
    d0, d1 = deg_raw[0, :N], deg_raw[1, :N]

    y1_ = pl.pallas_call(
        _prep_tc,
        grid=(NRB,),
        in_specs=[_row_spec(2 * HALF), _row_spec(HALF), _row_spec(HALF)],
        out_specs=_row_spec(2 * HALF),
        out_shape=jax.ShapeDtypeStruct((N, 2 * HALF), jnp.float32),
    )(x, d0, d1)

    a1 = _prop_kernel(y1_.reshape(2 * N, HALF), comb)

    z, y2_ = pl.pallas_call(
        _mid_tc,
        grid=(NRB,),
        in_specs=[_row_spec(2 * HALF), _row_spec(HALF), _row_spec(HALF),
                  _row_spec(HALF), _row_spec(HALF),
                  _full_spec((256, 512)), _full_spec((1, 512)),
                  _full_spec((512, 256))],
        out_specs=[_row_spec(2 * HALF), _row_spec(2 * HALF)],
        out_shape=(jax.ShapeDtypeStruct((N, 2 * HALF), jnp.float32),
                   jax.ShapeDtypeStruct((N, 2 * HALF), jnp.float32)),
    )(x, a1[0, :N], a1[1, :N], d0, d1, W1.T, b1.reshape(1, -1), W2.T)

    a2 = _prop_kernel(y2_.reshape(2 * N, HALF), comb)

    out = pl.pallas_call(
        _final_tc,
        grid=(NRB,),
        in_specs=[_row_spec(2 * HALF), _row_spec(HALF), _row_spec(HALF),
                  _row_spec(HALF), _row_spec(HALF), _full_spec((1, 256))],
        out_specs=_row_spec(2 * HALF),
        out_shape=jax.ShapeDtypeStruct((N, 2 * HALF), jnp.float32),
    )(z, a2[0, :N], a2[1, :N], d0, d1, b2.reshape(1, -1))

    return out
